# Initial kernel scaffold; baseline (speedup 1.0000x reference)
import jax
import jax.numpy as jnp
from jax.experimental import pallas as pl


def kernel(state, task_indicator, W_cx1_1, b_cx1_1, W_cx1_2, b_cx1_2, W_cx2_1, b_cx2_1, W_cx2_2, b_cx2_2, W_cx3_1, b_cx3_1, W_cx3_2, b_cx3_2, W_lin1, b_lin1, W_lin2, b_lin2, W_lin3, b_lin3, W_lin4, b_lin4):
    raise NotImplementedError("write your pallas kernel here")


# speedup vs baseline: 3.1496x; 3.1496x over previous
import jax
import jax.numpy as jnp
import numpy as np
from jax.experimental import pallas as pl

_MININT = np.int32(-2147483648)
_MAXPOS = np.int32(2147483647)


def _kwta(x, key_src, k, tri_bf16):
    skey = jax.lax.bitcast_convert_type(key_src + 0.0, jnp.int32)
    skey = jnp.where(skey < 0, skey ^ _MAXPOS, skey)

    def body(i, t_u):
        bit = jax.lax.shift_left(jnp.int32(1), jnp.int32(31) - i)
        cand = t_u | bit
        thr = cand ^ _MININT
        cnt = jnp.sum((skey >= thr).astype(jnp.int32), axis=1, keepdims=True)
        return jnp.where(cnt >= k, cand, t_u)

    t_u = jax.lax.fori_loop(0, 32, body, jnp.zeros_like(k))
    t_s = t_u ^ _MININT

    gt = skey > t_s
    c_gt = jnp.sum(gt.astype(jnp.int32), axis=1, keepdims=True)
    eq = skey == t_s
    cum_excl = jnp.dot(eq.astype(jnp.bfloat16), tri_bf16,
                       preferred_element_type=jnp.float32)
    keep = eq & (cum_excl < (k - c_gt).astype(jnp.float32))
    mask = (gt | keep) & (k > 0)
    return jnp.where(mask, x, x / 3.0)


def _body(a_ref, w11_ref, b11_ref, w12_ref, b12_ref,
          w21_ref, b21_ref, w22_ref, b22_ref,
          w31_ref, b31_ref, w32_ref, b32_ref,
          wl1_ref, bl1_ref, wl2_ref, bl2_ref,
          wl3_ref, bl3_ref, wl4_ref, bl4_ref,
          t1_ref, t2_ref, t3_ref, out_ref):
    f32 = jnp.float32
    a = a_ref[...]

    h1 = jnp.tanh(jnp.dot(a, w11_ref[...], preferred_element_type=f32)
                  + b11_ref[...])
    z1 = jnp.dot(h1.astype(jnp.bfloat16), w12_ref[...],
                 preferred_element_type=f32) + b12_ref[...]
    k1 = jnp.argmax(z1, axis=1).astype(jnp.int32)[:, None]

    h2 = jnp.tanh(jnp.dot(a, w21_ref[...], preferred_element_type=f32)
                  + b21_ref[...])
    z2 = jnp.dot(h2.astype(jnp.bfloat16), w22_ref[...],
                 preferred_element_type=f32) + b22_ref[...]
    k2 = jnp.argmax(z2, axis=1).astype(jnp.int32)[:, None]

    h3 = jnp.tanh(jnp.dot(a, w31_ref[...], preferred_element_type=f32)
                  + b31_ref[...])
    z3 = jnp.dot(h3.astype(jnp.bfloat16), w32_ref[...],
                 preferred_element_type=f32) + b32_ref[...]
    k3 = jnp.argmax(z3, axis=1).astype(jnp.int32)[:, None]

    x = (jnp.dot(a, wl1_ref[...], preferred_element_type=f32)
         + bl1_ref[...])
    x = _kwta(x, x, k1, t1_ref[...])
    x = jnp.dot(x.astype(jnp.bfloat16), wl2_ref[...],
                preferred_element_type=f32) + bl2_ref[...]
    x = _kwta(x, x, k2, t2_ref[...])
    x = jnp.dot(x.astype(jnp.bfloat16), wl3_ref[...],
                preferred_element_type=f32) + bl3_ref[...]
    col = jax.lax.broadcasted_iota(jnp.int32, x.shape, 1)
    key3 = jnp.where(col < 64, x, f32(-1e30))
    x = _kwta(x, key3, k3, t3_ref[...])
    out_ref[...] = jnp.dot(x.astype(jnp.bfloat16), wl4_ref[...],
                           preferred_element_type=f32) + bl4_ref[...]


def _tri(n):
    r = jnp.arange(n, dtype=jnp.int32)
    return (r[:, None] < r[None, :]).astype(jnp.bfloat16)


def _wt(W, kpad):
    o, fi = W.shape
    return jnp.pad(W.T, ((0, kpad - fi), (0, 0))).astype(jnp.bfloat16)


def kernel(state, task_indicator,
           W_cx1_1, b_cx1_1, W_cx1_2, b_cx1_2,
           W_cx2_1, b_cx2_1, W_cx2_2, b_cx2_2,
           W_cx3_1, b_cx3_1, W_cx3_2, b_cx3_2,
           W_lin1, b_lin1, W_lin2, b_lin2,
           W_lin3, b_lin3, W_lin4, b_lin4):
    B = state.shape[0]
    INP = state.shape[1] + task_indicator.shape[1]
    KPAD = ((INP + 127) // 128) * 128
    R = 256
    H2, H1, NH = 1024, 512, 64

    a = jnp.concatenate(
        [state, task_indicator,
         jnp.zeros((B, KPAD - INP), jnp.float32)], axis=1
    ).astype(jnp.bfloat16)

    w11 = _wt(W_cx1_1, KPAD)
    b11 = b_cx1_1[None, :]
    w21 = _wt(W_cx2_1, KPAD)
    b21 = b_cx2_1[None, :]
    w31 = jnp.pad(_wt(W_cx3_1, KPAD), ((0, 0), (0, 64)))
    b31 = jnp.pad(b_cx3_1, (0, 64))[None, :]
    wl1 = _wt(W_lin1, KPAD)
    bl1 = b_lin1[None, :]

    w12 = W_cx1_2.T.astype(jnp.bfloat16)
    b12 = b_cx1_2[None, :]
    w22 = W_cx2_2.T.astype(jnp.bfloat16)
    b22 = b_cx2_2[None, :]
    w32 = jnp.pad(W_cx3_2.T, ((0, 64), (0, 64))).astype(jnp.bfloat16)
    b32 = jnp.pad(b_cx3_2, (0, 64), constant_values=-1e9)[None, :]
    wl2 = W_lin2.T.astype(jnp.bfloat16)
    bl2 = b_lin2[None, :]
    wl3 = jnp.pad(W_lin3.T, ((0, 0), (0, 64))).astype(jnp.bfloat16)
    bl3 = jnp.pad(b_lin3, (0, 64))[None, :]
    wl4 = jnp.pad(W_lin4.T, ((0, 64), (0, 64))).astype(jnp.bfloat16)
    bl4 = jnp.pad(b_lin4, (0, 64))[None, :]

    t1, t2, t3 = _tri(H2), _tri(H1), _tri(128)

    def const(shape):
        return pl.BlockSpec(shape, lambda i: (0, 0))

    out = pl.pallas_call(
        _body,
        grid=(B // R,),
        in_specs=[
            pl.BlockSpec((R, KPAD), lambda i: (i, 0)),
            const(w11.shape), const(b11.shape), const(w12.shape), const(b12.shape),
            const(w21.shape), const(b21.shape), const(w22.shape), const(b22.shape),
            const(w31.shape), const(b31.shape), const(w32.shape), const(b32.shape),
            const(wl1.shape), const(bl1.shape), const(wl2.shape), const(bl2.shape),
            const(wl3.shape), const(bl3.shape), const(wl4.shape), const(bl4.shape),
            const(t1.shape), const(t2.shape), const(t3.shape),
        ],
        out_specs=pl.BlockSpec((R, 128), lambda i: (i, 0)),
        out_shape=jax.ShapeDtypeStruct((B, 128), jnp.float32),
    )(a, w11, b11, w12, b12, w21, b21, w22, b22, w31, b31, w32, b32,
      wl1, bl1, wl2, bl2, wl3, bl3, wl4, bl4, t1, t2, t3)
    return out[:, :NH]
